# baseline (speedup 1.0000x reference)
import functools

import jax
import jax.numpy as jnp
from jax import lax
from jax.experimental import pallas as pl
from jax.experimental.pallas import tpu as pltpu
from jax.experimental.pallas import tpu_sc as plsc

N, D, O, E, TOP_K = 4096, 1024, 1024, 8, 2
BMM = 256
NB = (TOP_K * N) // BMM + E
PADDED = NB * BMM
NC, NS, L = 2, 16, 16
NW = NC * NS
TPW = N // NW
DCH = 32
CH = 16


def _cumsum(x, axis):
    n = x.shape[axis]
    sh = 1
    while sh < n:
        if axis == 0:
            shifted = jnp.concatenate(
                [jnp.zeros((sh,) + x.shape[1:], x.dtype), x[:-sh]], axis=0)
        else:
            shifted = jnp.concatenate(
                [jnp.zeros(x.shape[:1] + (sh,), x.dtype), x[:, :-sh]], axis=1)
        x = x + shifted
        sh *= 2
    return x


def _pack16(a16, b16):
    a32 = lax.bitcast_convert_type(a16, jnp.uint16).astype(jnp.int32)
    b32 = lax.bitcast_convert_type(b16, jnp.uint16).astype(jnp.int32)
    return a32 | (b32 << 16)


def _unpack16(p):
    lo = lax.bitcast_convert_type(p << 16, jnp.float32)
    hi = lax.bitcast_convert_type(p & jnp.int32(-65536), jnp.float32)
    return lo, hi


def _gating_body(x_ref, wg_ref, bg_ref,
                 logits_ref, sw_ref, ti_ref, tv_ref,
                 pos0_ref, pos1_ref, bex_ref, w0_ref, w1_ref, xp_ref):
    xb16 = x_ref[...].astype(jnp.bfloat16)
    wg16 = wg_ref[...].astype(jnp.bfloat16)
    logits = jnp.dot(xb16, wg16, preferred_element_type=jnp.float32)
    logits = logits + bg_ref[...]
    logits_ref[...] = logits

    xp_ref[...] = _pack16(xb16[:, :D // 2], xb16[:, D // 2:])

    m = jnp.max(logits, axis=1, keepdims=True)
    p = jnp.exp(logits - m)
    probs = p / jnp.sum(p, axis=1, keepdims=True)

    iota = lax.broadcasted_iota(jnp.int32, (N, E), 1)
    v0 = jnp.max(probs, axis=1)
    i0 = jnp.min(jnp.where(probs == v0[:, None], iota, E), axis=1)
    mask0 = iota == i0[:, None]
    masked = jnp.where(mask0, -1.0, probs)
    v1 = jnp.max(masked, axis=1)
    i1 = jnp.min(jnp.where(masked == v1[:, None], iota, E), axis=1)
    mask1 = iota == i1[:, None]

    ti_ref[...] = jnp.concatenate([i0[:, None], i1[:, None]], axis=1)
    tv_ref[...] = jnp.concatenate([v0[:, None], v1[:, None]], axis=1)
    denom = v0 + v1 + 1e-8
    sw_ref[...] = jnp.where(mask0 | mask1, probs, 0.0) / denom[:, None]
    w0_ref[...] = (v0 / denom)[:, None]
    w1_ref[...] = (v1 / denom)[:, None]

    oh0 = mask0.astype(jnp.int32)
    oh1 = mask1.astype(jnp.int32)
    c0 = _cumsum(oh0, axis=0)
    c1 = _cumsum(oh1, axis=0)
    cnt0 = c0[-1:, :]
    counts = cnt0 + c1[-1:, :]
    blocks_e = (counts + BMM - 1) // BMM
    cumblk = _cumsum(blocks_e, axis=1)
    pad_off = (cumblk - blocks_e) * BMM
    pos0_ref[...] = jnp.sum((pad_off + c0 - oh0) * oh0, axis=1, keepdims=True)
    pos1_ref[...] = jnp.sum((pad_off + cnt0 + c1 - oh1) * oh1, axis=1,
                            keepdims=True)
    iota_nb = lax.broadcasted_iota(jnp.int32, (NB, E), 0)
    bex = jnp.sum((iota_nb >= cumblk).astype(jnp.int32), axis=1, keepdims=True)
    bex_ref[...] = jnp.minimum(bex, E - 1)


def _dispatch_body(x_hbm, p0_hbm, p1_hbm,
                   xg_hbm,
                   idx0_v, idx1_v, rows_v, sem_m, sem_s0, sem_s1):
    wid = lax.axis_index("s") * NC + lax.axis_index("c")
    base = wid * TPW
    NCHD = TPW // DCH
    meta = []
    for ci in range(NCHD):
        b = base + ci * DCH
        meta.append(pltpu.async_copy(p0_hbm.at[pl.ds(b, DCH)],
                                     idx0_v.at[ci], sem_m))
        meta.append(pltpu.async_copy(p1_hbm.at[pl.ds(b, DCH)],
                                     idx1_v.at[ci], sem_m))
    for cp in meta:
        cp.wait()
    scat_pend = {}
    for ci in range(NCHD):
        buf = ci % 2
        if buf in scat_pend:
            for cp in scat_pend.pop(buf):
                cp.wait()
        b = base + ci * DCH
        pltpu.sync_copy(x_hbm.at[pl.ds(b, DCH)], rows_v.at[buf])
        sem = sem_s0 if buf == 0 else sem_s1
        scat_pend[buf] = (
            pltpu.async_copy(rows_v.at[buf], xg_hbm.at[idx0_v.at[ci]], sem),
            pltpu.async_copy(rows_v.at[buf], xg_hbm.at[idx1_v.at[ci]], sem),
        )
    for cps in scat_pend.values():
        for cp in cps:
            cp.wait()


def _gmm_body(bex_sref, xg_ref, we_ref, be_ref, yg_ref):

    lo, hi = _unpack16(xg_ref[...])
    acc = jnp.dot(lo.astype(jnp.bfloat16), we_ref[0, :D // 2],
                  preferred_element_type=jnp.float32)
    acc = acc + jnp.dot(hi.astype(jnp.bfloat16), we_ref[0, D // 2:],
                        preferred_element_type=jnp.float32)
    acc = acc + be_ref[0]
    a16 = acc[:, :O // 2].astype(jnp.bfloat16)
    b16 = acc[:, O // 2:].astype(jnp.bfloat16)
    yg_ref[...] = _pack16(a16, b16)


def _combine_body(yg_hbm, p0_hbm, p1_hbm, w0_hbm, w1_hbm, pred_hbm,
                  idx0_v, idx1_v, w0_v, w1_v, buf0s, buf1s, out_v,
                  sem_m, sem_g0, sem_g1, sem_o):
    wid = lax.axis_index("s") * NC + lax.axis_index("c")
    base = wid * TPW
    NCH = TPW // CH

    meta = [
        pltpu.async_copy(p0_hbm.at[pl.ds(base, TPW)], idx0_v, sem_m),
        pltpu.async_copy(p1_hbm.at[pl.ds(base, TPW)], idx1_v, sem_m),
        pltpu.async_copy(w0_hbm.at[pl.ds(base, TPW)],
                         w0_v.at[pl.ds(0, TPW)], sem_m),
        pltpu.async_copy(w1_hbm.at[pl.ds(base, TPW)],
                         w1_v.at[pl.ds(0, TPW)], sem_m),
    ]
    for cp in meta:
        cp.wait()

    def start_gathers(ci, buf):
        sem = sem_g0 if buf == 0 else sem_g1
        sl = pl.ds(ci * CH, CH)
        c0 = pltpu.async_copy(yg_hbm.at[idx0_v.at[sl]], buf0s.at[buf], sem)
        c1 = pltpu.async_copy(yg_hbm.at[idx1_v.at[sl]], buf1s.at[buf], sem)
        return c0, c1

    pend = {0: start_gathers(0, 0)}
    out_pend = {}
    for ci in range(NCH):
        cur = ci % 2
        nxt = 1 - cur
        if ci + 1 < NCH:
            pend[nxt] = start_gathers(ci + 1, nxt)
        c0, c1 = pend.pop(cur)
        c0.wait()
        c1.wait()
        if (ci - 2) in out_pend:
            out_pend.pop(ci - 2).wait()

        def row_add(r, carry):
            w0s = w0_v[pl.ds(ci * CH + r, L)][0]
            w1s = w1_v[pl.ds(ci * CH + r, L)][0]
            for j in range(O // 2 // L):
                o = j * L
                v0 = buf0s[cur, r, pl.ds(o, L)]
                v1 = buf1s[cur, r, pl.ds(o, L)]
                lo0, hi0 = _unpack16(v0)
                lo1, hi1 = _unpack16(v1)
                out_v[cur, r, pl.ds(o, L)] = w0s * lo0 + w1s * lo1
                out_v[cur, r, pl.ds(O // 2 + o, L)] = w0s * hi0 + w1s * hi1
            return carry

        lax.fori_loop(0, CH, row_add, 0)
        out_pend[ci] = pltpu.async_copy(
            out_v.at[cur], pred_hbm.at[pl.ds(base + ci * CH, CH)], sem_o)
    for cp in out_pend.values():
        cp.wait()


_sc_mesh = plsc.VectorSubcoreMesh(core_axis_name="c", subcore_axis_name="s")

_dispatch = functools.partial(
    pl.kernel,
    out_type=jax.ShapeDtypeStruct((PADDED, D // 2), jnp.int32),
    mesh=_sc_mesh,
    scratch_types=[
        pltpu.VMEM((TPW // DCH, DCH), jnp.int32),
        pltpu.VMEM((TPW // DCH, DCH), jnp.int32),
        pltpu.VMEM((2, DCH, D // 2), jnp.int32),
        pltpu.SemaphoreType.DMA,
        pltpu.SemaphoreType.DMA,
        pltpu.SemaphoreType.DMA,
    ],
)(_dispatch_body)

_combine = functools.partial(
    pl.kernel,
    out_type=jax.ShapeDtypeStruct((N, O), jnp.float32),
    mesh=_sc_mesh,
    scratch_types=[
        pltpu.VMEM((TPW,), jnp.int32),
        pltpu.VMEM((TPW,), jnp.int32),
        pltpu.VMEM((TPW + L,), jnp.float32),
        pltpu.VMEM((TPW + L,), jnp.float32),
        pltpu.VMEM((2, CH, O // 2), jnp.int32),
        pltpu.VMEM((2, CH, O // 2), jnp.int32),
        pltpu.VMEM((2, CH, O), jnp.float32),
        pltpu.SemaphoreType.DMA,
        pltpu.SemaphoreType.DMA,
        pltpu.SemaphoreType.DMA,
        pltpu.SemaphoreType.DMA,
    ],
)(_combine_body)


@jax.jit
def kernel(x, Wg, bg, We, be):
    gout = pl.pallas_call(
        _gating_body,
        grid=(1,),
        in_specs=[
            pl.BlockSpec((N, D), lambda i: (0, 0)),
            pl.BlockSpec((D, E), lambda i: (0, 0)),
            pl.BlockSpec((1, E), lambda i: (0, 0)),
        ],
        out_specs=[
            pl.BlockSpec((N, E), lambda i: (0, 0)),
            pl.BlockSpec((N, E), lambda i: (0, 0)),
            pl.BlockSpec((N, TOP_K), lambda i: (0, 0)),
            pl.BlockSpec((N, TOP_K), lambda i: (0, 0)),
            pl.BlockSpec((N, 1), lambda i: (0, 0)),
            pl.BlockSpec((N, 1), lambda i: (0, 0)),
            pl.BlockSpec((NB, 1), lambda i: (0, 0)),
            pl.BlockSpec((N, 1), lambda i: (0, 0)),
            pl.BlockSpec((N, 1), lambda i: (0, 0)),
            pl.BlockSpec((N, D // 2), lambda i: (0, 0)),
        ],
        out_shape=[
            jax.ShapeDtypeStruct((N, E), jnp.float32),
            jax.ShapeDtypeStruct((N, E), jnp.float32),
            jax.ShapeDtypeStruct((N, TOP_K), jnp.int32),
            jax.ShapeDtypeStruct((N, TOP_K), jnp.float32),
            jax.ShapeDtypeStruct((N, 1), jnp.int32),
            jax.ShapeDtypeStruct((N, 1), jnp.int32),
            jax.ShapeDtypeStruct((NB, 1), jnp.int32),
            jax.ShapeDtypeStruct((N, 1), jnp.float32),
            jax.ShapeDtypeStruct((N, 1), jnp.float32),
            jax.ShapeDtypeStruct((N, D // 2), jnp.int32),
        ],
        compiler_params=pltpu.CompilerParams(
            dimension_semantics=("arbitrary",),
        ),
    )(x, Wg, bg.reshape(1, E))
    logits, sw, ti, tv, pos0, pos1, bex, w0, w1, xp = gout
    pos0 = pos0.reshape(N)
    pos1 = pos1.reshape(N)
    w0 = w0.reshape(N)
    w1 = w1.reshape(N)
    bex = bex.reshape(NB)

    xg = _dispatch(xp, pos0, pos1)

    We16 = We.astype(jnp.bfloat16)

    yg = pl.pallas_call(
        _gmm_body,
        grid_spec=pltpu.PrefetchScalarGridSpec(
            num_scalar_prefetch=1,
            grid=(NB,),
            in_specs=[
                pl.BlockSpec((BMM, D // 2), lambda i, bex_ref: (i, 0)),
                pl.BlockSpec((1, D, O), lambda i, bex_ref: (bex_ref[i], 0, 0)),
                pl.BlockSpec((1, 1, O), lambda i, bex_ref: (bex_ref[i], 0, 0)),
            ],
            out_specs=pl.BlockSpec((BMM, O // 2), lambda i, bex_ref: (i, 0)),
        ),
        out_shape=jax.ShapeDtypeStruct((PADDED, O // 2), jnp.int32),
        compiler_params=pltpu.CompilerParams(
            dimension_semantics=("arbitrary",),
        ),
    )(bex, xg, We16, be.reshape(E, 1, O))

    pred = _combine(yg, pos0, pos1, w0, w1)
    return (pred, logits, sw, ti, tv)


# speedup vs baseline: 1.0757x; 1.0757x over previous
import functools

import jax
import jax.numpy as jnp
from jax import lax
from jax.experimental import pallas as pl
from jax.experimental.pallas import tpu as pltpu
from jax.experimental.pallas import tpu_sc as plsc

N, D, O, E, TOP_K = 4096, 1024, 1024, 8, 2
BMM = 256
NB = (TOP_K * N) // BMM + E
PADDED = NB * BMM
NC, NS, L = 2, 16, 16
NW = NC * NS
TPW = N // NW
DCH = 32
CH = 16


def _cumsum(x, axis):
    n = x.shape[axis]
    sh = 1
    while sh < n:
        if axis == 0:
            shifted = jnp.concatenate(
                [jnp.zeros((sh,) + x.shape[1:], x.dtype), x[:-sh]], axis=0)
        else:
            shifted = jnp.concatenate(
                [jnp.zeros(x.shape[:1] + (sh,), x.dtype), x[:, :-sh]], axis=1)
        x = x + shifted
        sh *= 2
    return x


def _pack16(a16, b16):
    a32 = lax.bitcast_convert_type(a16, jnp.uint16).astype(jnp.int32)
    b32 = lax.bitcast_convert_type(b16, jnp.uint16).astype(jnp.int32)
    return a32 | (b32 << 16)


def _unpack16(p):
    lo = lax.bitcast_convert_type(p << 16, jnp.float32)
    hi = lax.bitcast_convert_type(p & jnp.int32(-65536), jnp.float32)
    return lo, hi


def _gating_body(x_ref, wg_ref, bg_ref,
                 logits_ref, sw_ref, ti_ref, tv_ref,
                 pos0_ref, pos1_ref, bex_ref, w0_ref, w1_ref, xp_ref):
    xb16 = x_ref[...].astype(jnp.bfloat16)
    wg16 = wg_ref[...].astype(jnp.bfloat16)
    logits = jnp.dot(xb16, wg16, preferred_element_type=jnp.float32)
    logits = logits + bg_ref[...]
    logits_ref[...] = logits

    xp_ref[...] = _pack16(xb16[:, :D // 2], xb16[:, D // 2:])

    m = jnp.max(logits, axis=1, keepdims=True)
    p = jnp.exp(logits - m)
    probs = p / jnp.sum(p, axis=1, keepdims=True)

    iota = lax.broadcasted_iota(jnp.int32, (N, E), 1)
    v0 = jnp.max(probs, axis=1)
    i0 = jnp.min(jnp.where(probs == v0[:, None], iota, E), axis=1)
    mask0 = iota == i0[:, None]
    masked = jnp.where(mask0, -1.0, probs)
    v1 = jnp.max(masked, axis=1)
    i1 = jnp.min(jnp.where(masked == v1[:, None], iota, E), axis=1)
    mask1 = iota == i1[:, None]

    ti_ref[...] = jnp.concatenate([i0[:, None], i1[:, None]], axis=1)
    tv_ref[...] = jnp.concatenate([v0[:, None], v1[:, None]], axis=1)
    denom = v0 + v1 + 1e-8
    sw_ref[...] = jnp.where(mask0 | mask1, probs, 0.0) / denom[:, None]
    w0_ref[...] = v0 / denom
    w1_ref[...] = v1 / denom

    oh0 = mask0.astype(jnp.int32)
    oh1 = mask1.astype(jnp.int32)
    c0 = _cumsum(oh0, axis=0)
    c1 = _cumsum(oh1, axis=0)
    cnt0 = c0[-1:, :]
    counts = cnt0 + c1[-1:, :]
    blocks_e = (counts + BMM - 1) // BMM
    cumblk = _cumsum(blocks_e, axis=1)
    pad_off = (cumblk - blocks_e) * BMM
    pos0_ref[...] = jnp.sum((pad_off + c0 - oh0) * oh0, axis=1)
    pos1_ref[...] = jnp.sum((pad_off + cnt0 + c1 - oh1) * oh1, axis=1)
    iota_nb = lax.broadcasted_iota(jnp.int32, (NB, E), 0)
    bex = jnp.sum((iota_nb >= cumblk).astype(jnp.int32), axis=1)
    bex_ref[...] = jnp.minimum(bex, E - 1)


def _dispatch_body(x_hbm, p0_hbm, p1_hbm,
                   xg_hbm,
                   idx0_v, idx1_v, rows_v, sem_m, sem_s0, sem_s1):
    wid = lax.axis_index("s") * NC + lax.axis_index("c")
    base = wid * TPW
    NCHD = TPW // DCH
    meta = []
    for ci in range(NCHD):
        b = base + ci * DCH
        meta.append(pltpu.async_copy(p0_hbm.at[pl.ds(b, DCH)],
                                     idx0_v.at[ci], sem_m))
        meta.append(pltpu.async_copy(p1_hbm.at[pl.ds(b, DCH)],
                                     idx1_v.at[ci], sem_m))
    for cp in meta:
        cp.wait()
    scat_pend = {}
    for ci in range(NCHD):
        buf = ci % 2
        if buf in scat_pend:
            for cp in scat_pend.pop(buf):
                cp.wait()
        b = base + ci * DCH
        pltpu.sync_copy(x_hbm.at[pl.ds(b, DCH)], rows_v.at[buf])
        sem = sem_s0 if buf == 0 else sem_s1
        scat_pend[buf] = (
            pltpu.async_copy(rows_v.at[buf], xg_hbm.at[idx0_v.at[ci]], sem),
            pltpu.async_copy(rows_v.at[buf], xg_hbm.at[idx1_v.at[ci]], sem),
        )
    for cps in scat_pend.values():
        for cp in cps:
            cp.wait()


def _gmm_body(bex_sref, xg_ref, we_ref, be_ref, yg_ref, we16_s):
    i = pl.program_id(0)
    prev = bex_sref[jnp.maximum(i - 1, 0)]

    @pl.when((i == 0) | (bex_sref[i] != prev))
    def _():
        we16_s[...] = we_ref[0].astype(jnp.bfloat16)

    lo, hi = _unpack16(xg_ref[...])
    acc = jnp.dot(lo.astype(jnp.bfloat16), we16_s[:D // 2],
                  preferred_element_type=jnp.float32)
    acc = acc + jnp.dot(hi.astype(jnp.bfloat16), we16_s[D // 2:],
                        preferred_element_type=jnp.float32)
    acc = acc + be_ref[0]
    a16 = acc[:, :O // 2].astype(jnp.bfloat16)
    b16 = acc[:, O // 2:].astype(jnp.bfloat16)
    yg_ref[...] = _pack16(a16, b16)


def _combine_body(yg_hbm, p0_hbm, p1_hbm, w0_hbm, w1_hbm, pred_hbm,
                  idx0_v, idx1_v, w0_v, w1_v, buf0s, buf1s, out_v,
                  sem_m, sem_g0, sem_g1, sem_o):
    wid = lax.axis_index("s") * NC + lax.axis_index("c")
    base = wid * TPW
    NCH = TPW // CH

    meta = [
        pltpu.async_copy(p0_hbm.at[pl.ds(base, TPW)], idx0_v, sem_m),
        pltpu.async_copy(p1_hbm.at[pl.ds(base, TPW)], idx1_v, sem_m),
        pltpu.async_copy(w0_hbm.at[pl.ds(base, TPW)],
                         w0_v.at[pl.ds(0, TPW)], sem_m),
        pltpu.async_copy(w1_hbm.at[pl.ds(base, TPW)],
                         w1_v.at[pl.ds(0, TPW)], sem_m),
    ]
    for cp in meta:
        cp.wait()

    def start_gathers(ci, buf):
        sem = sem_g0 if buf == 0 else sem_g1
        sl = pl.ds(ci * CH, CH)
        c0 = pltpu.async_copy(yg_hbm.at[idx0_v.at[sl]], buf0s.at[buf], sem)
        c1 = pltpu.async_copy(yg_hbm.at[idx1_v.at[sl]], buf1s.at[buf], sem)
        return c0, c1

    pend = {0: start_gathers(0, 0)}
    out_pend = {}
    for ci in range(NCH):
        cur = ci % 2
        nxt = 1 - cur
        if ci + 1 < NCH:
            pend[nxt] = start_gathers(ci + 1, nxt)
        c0, c1 = pend.pop(cur)
        c0.wait()
        c1.wait()
        if (ci - 2) in out_pend:
            out_pend.pop(ci - 2).wait()

        def row_add(r, carry):
            w0s = w0_v[pl.ds(ci * CH + r, L)][0]
            w1s = w1_v[pl.ds(ci * CH + r, L)][0]
            for j in range(O // 2 // L):
                o = j * L
                v0 = buf0s[cur, r, pl.ds(o, L)]
                v1 = buf1s[cur, r, pl.ds(o, L)]
                lo0, hi0 = _unpack16(v0)
                lo1, hi1 = _unpack16(v1)
                out_v[cur, r, pl.ds(o, L)] = w0s * lo0 + w1s * lo1
                out_v[cur, r, pl.ds(O // 2 + o, L)] = w0s * hi0 + w1s * hi1
            return carry

        lax.fori_loop(0, CH, row_add, 0)
        out_pend[ci] = pltpu.async_copy(
            out_v.at[cur], pred_hbm.at[pl.ds(base + ci * CH, CH)], sem_o)
    for cp in out_pend.values():
        cp.wait()


_sc_mesh = plsc.VectorSubcoreMesh(core_axis_name="c", subcore_axis_name="s")

_dispatch = functools.partial(
    pl.kernel,
    out_type=jax.ShapeDtypeStruct((PADDED, D // 2), jnp.int32),
    mesh=_sc_mesh,
    scratch_types=[
        pltpu.VMEM((TPW // DCH, DCH), jnp.int32),
        pltpu.VMEM((TPW // DCH, DCH), jnp.int32),
        pltpu.VMEM((2, DCH, D // 2), jnp.int32),
        pltpu.SemaphoreType.DMA,
        pltpu.SemaphoreType.DMA,
        pltpu.SemaphoreType.DMA,
    ],
)(_dispatch_body)

_combine = functools.partial(
    pl.kernel,
    out_type=jax.ShapeDtypeStruct((N, O), jnp.float32),
    mesh=_sc_mesh,
    scratch_types=[
        pltpu.VMEM((TPW,), jnp.int32),
        pltpu.VMEM((TPW,), jnp.int32),
        pltpu.VMEM((TPW + L,), jnp.float32),
        pltpu.VMEM((TPW + L,), jnp.float32),
        pltpu.VMEM((2, CH, O // 2), jnp.int32),
        pltpu.VMEM((2, CH, O // 2), jnp.int32),
        pltpu.VMEM((2, CH, O), jnp.float32),
        pltpu.SemaphoreType.DMA,
        pltpu.SemaphoreType.DMA,
        pltpu.SemaphoreType.DMA,
        pltpu.SemaphoreType.DMA,
    ],
)(_combine_body)


@jax.jit
def kernel(x, Wg, bg, We, be):
    gout = pl.pallas_call(
        _gating_body,
        grid=(1,),
        in_specs=[
            pl.BlockSpec((N, D), lambda i: (0, 0)),
            pl.BlockSpec((D, E), lambda i: (0, 0)),
            pl.BlockSpec((1, E), lambda i: (0, 0)),
        ],
        out_specs=[
            pl.BlockSpec((N, E), lambda i: (0, 0)),
            pl.BlockSpec((N, E), lambda i: (0, 0)),
            pl.BlockSpec((N, TOP_K), lambda i: (0, 0)),
            pl.BlockSpec((N, TOP_K), lambda i: (0, 0)),
            pl.BlockSpec((N,), lambda i: (0,)),
            pl.BlockSpec((N,), lambda i: (0,)),
            pl.BlockSpec((NB,), lambda i: (0,)),
            pl.BlockSpec((N,), lambda i: (0,)),
            pl.BlockSpec((N,), lambda i: (0,)),
            pl.BlockSpec((N, D // 2), lambda i: (0, 0)),
        ],
        out_shape=[
            jax.ShapeDtypeStruct((N, E), jnp.float32),
            jax.ShapeDtypeStruct((N, E), jnp.float32),
            jax.ShapeDtypeStruct((N, TOP_K), jnp.int32),
            jax.ShapeDtypeStruct((N, TOP_K), jnp.float32),
            jax.ShapeDtypeStruct((N,), jnp.int32),
            jax.ShapeDtypeStruct((N,), jnp.int32),
            jax.ShapeDtypeStruct((NB,), jnp.int32),
            jax.ShapeDtypeStruct((N,), jnp.float32),
            jax.ShapeDtypeStruct((N,), jnp.float32),
            jax.ShapeDtypeStruct((N, D // 2), jnp.int32),
        ],
        compiler_params=pltpu.CompilerParams(
            dimension_semantics=("arbitrary",),
        ),
    )(x, Wg, bg.reshape(1, E))
    logits, sw, ti, tv, pos0, pos1, bex, w0, w1, xp = gout

    xg = _dispatch(xp, pos0, pos1)

    yg = pl.pallas_call(
        _gmm_body,
        grid_spec=pltpu.PrefetchScalarGridSpec(
            num_scalar_prefetch=1,
            grid=(NB,),
            in_specs=[
                pl.BlockSpec((BMM, D // 2), lambda i, bex_ref: (i, 0)),
                pl.BlockSpec((1, D, O), lambda i, bex_ref: (bex_ref[i], 0, 0)),
                pl.BlockSpec((1, 1, O), lambda i, bex_ref: (bex_ref[i], 0, 0)),
            ],
            out_specs=pl.BlockSpec((BMM, O // 2), lambda i, bex_ref: (i, 0)),
            scratch_shapes=[pltpu.VMEM((D, O), jnp.bfloat16)],
        ),
        out_shape=jax.ShapeDtypeStruct((PADDED, O // 2), jnp.int32),
        compiler_params=pltpu.CompilerParams(
            dimension_semantics=("arbitrary",),
        ),
    )(bex, xg, We, be.reshape(E, 1, O))

    pred = _combine(yg, pos0, pos1, w0, w1)
    return (pred, logits, sw, ti, tv)
